# baseline (speedup 1.0000x reference)
import jax
import jax.numpy as jnp
from jax import lax
from jax.experimental import pallas as pl
from jax.experimental.pallas import tpu as pltpu
from jax.experimental.pallas import tpu_sc as plsc

N = 1000
B = 16384
NC = 2
NS = 16
NW = NC * NS
BPW = B // NW
LANES = 16
CHUNK = 128
NCHUNK = BPW // CHUNK
IROWS = 2 * BPW // CHUNK

def _lane_shuffle(v, idx):
    return lax.gather(
        v, idx[:, None],
        lax.GatherDimensionNumbers(
            offset_dims=(), collapsed_slice_dims=(0,), start_index_map=(0,)),
        (1,),
        mode=lax.GatherScatterMode.PROMISE_IN_BOUNDS)


def _lse_fold_body(wc_ref, wa_ref, t2_ref):
    x = wc_ref[...]
    m = jnp.max(x, axis=1, keepdims=True)
    s = jnp.sum(jnp.exp(x - m), axis=1, keepdims=True)
    lse_c = m + jnp.log(s)
    wa = wa_ref[...]
    ma = jnp.max(wa)
    sa = jnp.sum(jnp.exp(wa - ma))
    lse_a = ma + jnp.log(sa)
    t2_ref[...] = x + (wa - lse_a - lse_c)


def _sc_body(in_hbm, t2_hbm, out_hbm, iv_v, idx_v, out_v, sem, gsem):
    wid = lax.axis_index("s") * NC + lax.axis_index("c")

    pltpu.async_copy(in_hbm.at[pl.ds(wid * IROWS, IROWS)], iv_v, sem).wait()

    lane = lax.iota(jnp.int32, LANES)
    rot1 = lax.bitwise_and(lane + 1, LANES - 1)
    compact = lax.bitwise_and(lane * 2, LANES - 1)
    low_half = lane < (LANES // 2)

    for i in range(BPW // LANES):
        q1, t1 = (2 * i) // 8, (2 * i) % 8
        q2, t2 = (2 * i + 1) // 8, (2 * i + 1) % 8
        v1 = iv_v[q1, pl.ds(t1 * LANES, LANES)]
        v2 = iv_v[q2, pl.ds(t2 * LANES, LANES)]
        u1 = v1 * N + _lane_shuffle(v1, rot1)
        u2 = v2 * N + _lane_shuffle(v2, rot1)
        flat = jnp.where(low_half,
                         _lane_shuffle(u1, compact),
                         _lane_shuffle(u2, compact))
        idx_v[i // 8, pl.ds((i % 8) * LANES, LANES)] = flat

    gathers = [
        pltpu.async_copy(t2_hbm.at[idx_v.at[j]], out_v.at[j], gsem)
        for j in range(NCHUNK)
    ]
    for cp in gathers:
        cp.wait()

    pltpu.sync_copy(out_v, out_hbm.at[pl.ds(wid * NCHUNK, NCHUNK)])


@jax.jit
def kernel(inputs, w_A, w_cond):
    inputs = inputs.astype(jnp.int32)
    w_A = w_A.astype(jnp.float32)
    w_cond = w_cond.astype(jnp.float32)

    table2 = pl.pallas_call(
        _lse_fold_body,
        out_shape=jax.ShapeDtypeStruct((N, N), jnp.float32),
    )(w_cond, w_A[:, None])

    in2 = inputs.reshape(2 * B // CHUNK, CHUNK)
    t2_flat = table2.reshape(N * N)

    sc_kernel = pl.kernel(
        _sc_body,
        out_type=jax.ShapeDtypeStruct((B // CHUNK, CHUNK), jnp.float32),
        mesh=plsc.VectorSubcoreMesh(core_axis_name="c", subcore_axis_name="s"),
        scratch_types=[
            pltpu.VMEM((IROWS, CHUNK), jnp.int32),
            pltpu.VMEM((NCHUNK, CHUNK), jnp.int32),
            pltpu.VMEM((NCHUNK, CHUNK), jnp.float32),
            pltpu.SemaphoreType.DMA,
            pltpu.SemaphoreType.DMA,
        ],
    )
    out2 = sc_kernel(in2, t2_flat)
    return out2.reshape(B)


# speedup vs baseline: 1.0419x; 1.0419x over previous
import jax
import jax.numpy as jnp
from jax import lax
from jax.experimental import pallas as pl
from jax.experimental.pallas import tpu as pltpu
from jax.experimental.pallas import tpu_sc as plsc

N = 1000
NPAD = 1024
B = 16384
NC = 2
NS = 16
NW = NC * NS
BPW = B // NW
LANES = 16
CHUNK = 128
NCHUNK = BPW // CHUNK
IROWS = 2 * BPW // CHUNK

def _lane_shuffle(v, idx):
    return lax.gather(
        v, idx[:, None],
        lax.GatherDimensionNumbers(
            offset_dims=(), collapsed_slice_dims=(0,), start_index_map=(0,)),
        (1,),
        mode=lax.GatherScatterMode.PROMISE_IN_BOUNDS)


def _lse_fold_body(wc_ref, wa_ref, t2_ref):
    x = wc_ref[...]
    m = jnp.max(x, axis=1, keepdims=True)
    s = jnp.sum(jnp.exp(x - m), axis=1, keepdims=True)
    lse_c = m + jnp.log(s)
    wa = wa_ref[...]
    ma = jnp.max(wa)
    sa = jnp.sum(jnp.exp(wa - ma))
    lse_a = ma + jnp.log(sa)
    t2_ref[:, :N] = x + (wa - lse_a - lse_c)


def _sc_body(in_hbm, t2_hbm, out_hbm, iv_v, idx_v, out_v, sem, gsem):
    wid = lax.axis_index("s") * NC + lax.axis_index("c")

    pltpu.async_copy(in_hbm.at[pl.ds(wid * IROWS, IROWS)], iv_v, sem).wait()

    lane = lax.iota(jnp.int32, LANES)
    rot1 = lax.bitwise_and(lane + 1, LANES - 1)
    compact = lax.bitwise_and(lane * 2, LANES - 1)
    low_half = lane < (LANES // 2)

    for i in range(BPW // LANES):
        q1, t1 = (2 * i) // 8, (2 * i) % 8
        q2, t2 = (2 * i + 1) // 8, (2 * i + 1) % 8
        v1 = iv_v[q1, pl.ds(t1 * LANES, LANES)]
        v2 = iv_v[q2, pl.ds(t2 * LANES, LANES)]
        u1 = v1 * NPAD + _lane_shuffle(v1, rot1)
        u2 = v2 * NPAD + _lane_shuffle(v2, rot1)
        flat = jnp.where(low_half,
                         _lane_shuffle(u1, compact),
                         _lane_shuffle(u2, compact))
        idx_v[i // 8, pl.ds((i % 8) * LANES, LANES)] = flat

    gathers = [
        pltpu.async_copy(t2_hbm.at[idx_v.at[j]], out_v.at[j], gsem)
        for j in range(NCHUNK)
    ]
    for cp in gathers:
        cp.wait()

    pltpu.sync_copy(out_v, out_hbm.at[pl.ds(wid * NCHUNK, NCHUNK)])


@jax.jit
def kernel(inputs, w_A, w_cond):
    inputs = inputs.astype(jnp.int32)
    w_A = w_A.astype(jnp.float32)
    w_cond = w_cond.astype(jnp.float32)

    table2 = pl.pallas_call(
        _lse_fold_body,
        out_shape=jax.ShapeDtypeStruct((N, NPAD), jnp.float32),
    )(w_cond, w_A[:, None])

    in2 = inputs.reshape(2 * B // CHUNK, CHUNK)
    t2_flat = table2.reshape(N * NPAD)

    sc_kernel = pl.kernel(
        _sc_body,
        out_type=jax.ShapeDtypeStruct((B // CHUNK, CHUNK), jnp.float32),
        mesh=plsc.VectorSubcoreMesh(core_axis_name="c", subcore_axis_name="s"),
        scratch_types=[
            pltpu.VMEM((IROWS, CHUNK), jnp.int32),
            pltpu.VMEM((NCHUNK, CHUNK), jnp.int32),
            pltpu.VMEM((NCHUNK, CHUNK), jnp.float32),
            pltpu.SemaphoreType.DMA,
            pltpu.SemaphoreType.DMA,
        ],
    )
    out2 = sc_kernel(in2, t2_flat)
    return out2.reshape(B)
